# baseline (speedup 1.0000x reference)
import jax
import jax.numpy as jnp
from jax.experimental import pallas as pl

_DIM = 32
_NUM_Q = 8
_K = 1024
_COMMIT = 0.25
_BLOCK_B = 1024


def _rvq_body(x_ref, w_ref, wt_ref, w2_ref, quant_ref, idx_ref, loss_ref):
    r = x_ref[...]
    qs = []
    idxs = []
    part = jnp.zeros((), jnp.float32)
    for s in range(_NUM_Q):
        wt = wt_ref[s]
        m = jnp.dot(r.astype(jnp.bfloat16), wt.astype(jnp.bfloat16),
                    preferred_element_type=jnp.float32)
        w2 = w2_ref[s]
        r2 = jnp.sum(r * r, axis=1, keepdims=True)
        score = (r2 + w2) - 2.0 * m
        idx = jnp.argmin(score, axis=1)
        onehot = (jax.lax.broadcasted_iota(jnp.int32, score.shape, 1)
                  == idx[:, None]).astype(jnp.float32)
        q = jnp.dot(onehot, w_ref[s], preferred_element_type=jnp.float32,
                    precision=jax.lax.Precision.HIGHEST)
        qs.append(r + (q - r))
        idxs.append(idx)
        r = r - q
        c = _COMMIT if s < _NUM_Q - 1 else 1.0 + _COMMIT
        part = part + c * jnp.sum(r * r)
    quant_ref[...] = jnp.stack(qs, axis=1)
    idx_ref[...] = jnp.stack(idxs, axis=1).astype(jnp.int32)

    @pl.when(pl.program_id(0) == 0)
    def _():
        loss_ref[...] = jnp.zeros((1, 1), jnp.float32)

    loss_ref[...] = loss_ref[...] + part.reshape(1, 1)


def kernel(x, W):
    B, D = x.shape
    Q, K, _ = W.shape
    Wt = jnp.swapaxes(W, 1, 2)
    W2 = jnp.sum(W * W, axis=2)[:, None, :]
    n_blocks = B // _BLOCK_B
    quant, idx, loss = pl.pallas_call(
        _rvq_body,
        grid=(n_blocks,),
        in_specs=[
            pl.BlockSpec((_BLOCK_B, D), lambda i: (i, 0)),
            pl.BlockSpec((Q, K, D), lambda i: (0, 0, 0)),
            pl.BlockSpec((Q, D, K), lambda i: (0, 0, 0)),
            pl.BlockSpec((Q, 1, K), lambda i: (0, 0, 0)),
        ],
        out_specs=[
            pl.BlockSpec((_BLOCK_B, Q, D), lambda i: (i, 0, 0)),
            pl.BlockSpec((_BLOCK_B, Q), lambda i: (i, 0)),
            pl.BlockSpec((1, 1), lambda i: (0, 0)),
        ],
        out_shape=[
            jax.ShapeDtypeStruct((B, Q, D), jnp.float32),
            jax.ShapeDtypeStruct((B, Q), jnp.int32),
            jax.ShapeDtypeStruct((1, 1), jnp.float32),
        ],
    )(x, W, Wt, W2)
    return quant, idx, loss[0, 0] / (B * D)


# speedup vs baseline: 3.1592x; 3.1592x over previous
import jax
import jax.numpy as jnp
from jax.experimental import pallas as pl

_DIM = 32
_NUM_Q = 8
_K = 1024
_COMMIT = 0.25
_BLOCK_B = 1024


def _split_bf16_exact(w):
    def trunc(v):
        bits = jax.lax.bitcast_convert_type(v, jnp.uint32)
        return jax.lax.bitcast_convert_type(bits & jnp.uint32(0xFFFF0000),
                                            jnp.float32)
    a = trunc(w)
    rem1 = w - a
    b = trunc(rem1)
    c = rem1 - b
    return a, b, c


def _rvq_body(x_ref, wsplit_ref, wt_ref, w2_ref, quant_ref, idx_ref, loss_ref):
    r = x_ref[...]
    qs = []
    idxs = []
    part = jnp.zeros((), jnp.float32)
    for s in range(_NUM_Q):
        wt = wt_ref[s]
        m = jnp.dot(r.astype(jnp.bfloat16), wt.astype(jnp.bfloat16),
                    preferred_element_type=jnp.float32)
        w2 = w2_ref[s]
        r2 = jnp.sum(r * r, axis=1, keepdims=True)
        score = (r2 + w2) - 2.0 * m
        idx = jnp.argmin(score, axis=1)
        onehot = (jax.lax.broadcasted_iota(jnp.int32, score.shape, 1)
                  == idx[:, None]).astype(jnp.bfloat16)
        qcat = jnp.dot(onehot, wsplit_ref[s],
                       preferred_element_type=jnp.float32)
        q = (qcat[:, :_DIM] + qcat[:, _DIM:2 * _DIM]) + qcat[:, 2 * _DIM:]
        qs.append(r + (q - r))
        idxs.append(idx)
        r = r - q
        c = _COMMIT if s < _NUM_Q - 1 else 1.0 + _COMMIT
        part = part + c * jnp.sum(r * r)
    quant_ref[...] = jnp.stack(qs, axis=1)
    idx_ref[...] = jnp.stack(idxs, axis=1).astype(jnp.int32)

    @pl.when(pl.program_id(0) == 0)
    def _():
        loss_ref[...] = jnp.zeros((1, 1), jnp.float32)

    loss_ref[...] = loss_ref[...] + part.reshape(1, 1)


def kernel(x, W):
    B, D = x.shape
    Q, K, _ = W.shape
    Wt = jnp.swapaxes(W, 1, 2)
    W2 = jnp.sum(W * W, axis=2)[:, None, :]
    wa, wb, wc = _split_bf16_exact(W)
    Wsplit = jnp.concatenate([wa, wb, wc], axis=2).astype(jnp.bfloat16)
    n_blocks = B // _BLOCK_B
    quant, idx, loss = pl.pallas_call(
        _rvq_body,
        grid=(n_blocks,),
        in_specs=[
            pl.BlockSpec((_BLOCK_B, D), lambda i: (i, 0)),
            pl.BlockSpec((Q, K, 3 * D), lambda i: (0, 0, 0)),
            pl.BlockSpec((Q, D, K), lambda i: (0, 0, 0)),
            pl.BlockSpec((Q, 1, K), lambda i: (0, 0, 0)),
        ],
        out_specs=[
            pl.BlockSpec((_BLOCK_B, Q, D), lambda i: (i, 0, 0)),
            pl.BlockSpec((_BLOCK_B, Q), lambda i: (i, 0)),
            pl.BlockSpec((1, 1), lambda i: (0, 0)),
        ],
        out_shape=[
            jax.ShapeDtypeStruct((B, Q, D), jnp.float32),
            jax.ShapeDtypeStruct((B, Q), jnp.int32),
            jax.ShapeDtypeStruct((1, 1), jnp.float32),
        ],
    )(x, Wsplit, Wt, W2)
    return quant, idx, loss[0, 0] / (B * D)
